# Initial kernel scaffold; baseline (speedup 1.0000x reference)
import jax
import jax.numpy as jnp
from jax.experimental import pallas as pl


def kernel(x, edge_index, W1, b1, W2, b2):
    raise NotImplementedError("write your pallas kernel here")


# speedup vs baseline: 10.7383x; 10.7383x over previous
import functools
import jax
import jax.numpy as jnp
from jax import lax
from jax.experimental import pallas as pl
from jax.experimental.pallas import tpu as pltpu
from jax.experimental.pallas import tpu_sc as plsc

N_NODES = 10000
D = 128
N_PAD = 10240
DUMMY_DST = 10100
NC, NS = 2, 16
NW = NC * NS
K = 128
ROWS_PER_TILE = N_PAD // NS
DEG_W = 16

_sc_mesh = plsc.VectorSubcoreMesh(core_axis_name="c", subcore_axis_name="s")


def _sc_degree(dst_pad, ones_blk, zeros_blk, n_chunks):
    per_tile = n_chunks * K

    @functools.partial(
        pl.kernel,
        mesh=_sc_mesh,
        out_type=jax.ShapeDtypeStruct((NC, N_PAD, DEG_W), jnp.float32),
        scratch_types=[
            pltpu.VMEM((K,), jnp.int32),
            pltpu.VMEM((K, DEG_W), jnp.float32),
            pltpu.VMEM_SHARED((N_PAD, DEG_W), jnp.float32),
        ],
    )
    def k(dst_hbm, ones_hbm, zeros_hbm, out_hbm, dst_v, ones_v, acc_sh):
        c = lax.axis_index("c")
        s = lax.axis_index("s")
        pltpu.sync_copy(zeros_hbm, acc_sh.at[pl.ds(s * ROWS_PER_TILE, ROWS_PER_TILE)])
        pltpu.sync_copy(ones_hbm, ones_v)
        plsc.subcore_barrier()
        base = (c * NS + s) * per_tile

        def body(ch, _):
            pltpu.sync_copy(dst_hbm.at[pl.ds(base + ch * K, K)], dst_v)
            pltpu.sync_copy(ones_v, acc_sh.at[dst_v], add=True)
            return ()

        lax.fori_loop(0, n_chunks, body, ())
        plsc.subcore_barrier()
        pltpu.sync_copy(
            acc_sh.at[pl.ds(s * ROWS_PER_TILE, ROWS_PER_TILE)],
            out_hbm.at[c, pl.ds(s * ROWS_PER_TILE, ROWS_PER_TILE)],
        )

    return k(dst_pad, ones_blk, zeros_blk)


def _sc_scatter(src_pad, dst_pad, g, zeros_blk, n_chunks):
    per_tile = n_chunks * K

    @functools.partial(
        pl.kernel,
        mesh=_sc_mesh,
        out_type=jax.ShapeDtypeStruct((NC, N_PAD, D), jnp.float32),
        scratch_types=[
            pltpu.VMEM((K,), jnp.int32),
            pltpu.VMEM((K,), jnp.int32),
            pltpu.VMEM((K, D), jnp.float32),
            pltpu.VMEM_SHARED((N_PAD, D), jnp.float32),
            pltpu.SemaphoreType.DMA,
        ],
    )
    def k(src_hbm, dst_hbm, g_hbm, zeros_hbm, out_hbm, src_v, dst_v, rows_v, acc_sh, sem):
        c = lax.axis_index("c")
        s = lax.axis_index("s")
        pltpu.sync_copy(zeros_hbm, acc_sh.at[pl.ds(s * ROWS_PER_TILE, ROWS_PER_TILE)])
        plsc.subcore_barrier()
        base = (c * NS + s) * per_tile

        def body(ch, _):
            pltpu.sync_copy(src_hbm.at[pl.ds(base + ch * K, K)], src_v)
            pltpu.async_copy(g_hbm.at[src_v], rows_v, sem).wait()
            pltpu.sync_copy(dst_hbm.at[pl.ds(base + ch * K, K)], dst_v)
            pltpu.sync_copy(rows_v, acc_sh.at[dst_v], add=True)
            return ()

        lax.fori_loop(0, n_chunks, body, ())
        plsc.subcore_barrier()
        pltpu.sync_copy(
            acc_sh.at[pl.ds(s * ROWS_PER_TILE, ROWS_PER_TILE)],
            out_hbm.at[c, pl.ds(s * ROWS_PER_TILE, ROWS_PER_TILE)],
        )

    return k(src_pad, dst_pad, g, zeros_blk)


_BLK = 2048


def _tc_prologue(deg_parts, x_pad, W1):
    def body(degp, x_blk, w, g_out, dinv_out):
        deg = 1.0 + degp[0, :, 0] + degp[1, :, 0]
        dinv = lax.rsqrt(deg)
        h = jnp.dot(x_blk[...], w[...], preferred_element_type=jnp.float32)
        g_out[...] = h * dinv[:, None]
        dinv_out[...] = dinv

    return pl.pallas_call(
        body,
        grid=(N_PAD // _BLK,),
        in_specs=[
            pl.BlockSpec((NC, _BLK, DEG_W), lambda i: (0, i, 0)),
            pl.BlockSpec((_BLK, D), lambda i: (i, 0)),
            pl.BlockSpec((D, D), lambda i: (0, 0)),
        ],
        out_specs=[
            pl.BlockSpec((_BLK, D), lambda i: (i, 0)),
            pl.BlockSpec((_BLK,), lambda i: (i,)),
        ],
        out_shape=[
            jax.ShapeDtypeStruct((N_PAD, D), jnp.float32),
            jax.ShapeDtypeStruct((N_PAD,), jnp.float32),
        ],
    )(deg_parts, x_pad, W1)


def _tc_mid(s_parts, g1, dinv, b1, W2):
    def body(sp, g_blk, dinv_blk, b, w, g2_out):
        dinv = dinv_blk[...]
        z = (sp[0] + sp[1] + g_blk[...]) * dinv[:, None] + b[...][None, :]
        a = jnp.maximum(z, 0.0)
        h2 = jnp.dot(a, w[...], preferred_element_type=jnp.float32)
        g2_out[...] = h2 * dinv[:, None]

    return pl.pallas_call(
        body,
        grid=(N_PAD // _BLK,),
        in_specs=[
            pl.BlockSpec((NC, _BLK, D), lambda i: (0, i, 0)),
            pl.BlockSpec((_BLK, D), lambda i: (i, 0)),
            pl.BlockSpec((_BLK,), lambda i: (i,)),
            pl.BlockSpec((D,), lambda i: (0,)),
            pl.BlockSpec((D, D), lambda i: (0, 0)),
        ],
        out_specs=pl.BlockSpec((_BLK, D), lambda i: (i, 0)),
        out_shape=jax.ShapeDtypeStruct((N_PAD, D), jnp.float32),
    )(s_parts, g1, dinv, b1, W2)


def _tc_epilogue(s_parts, g2, dinv, b2):
    def body(sp, g_blk, dinv_blk, b, out):
        z = (sp[0] + sp[1] + g_blk[...]) * dinv_blk[...][:, None] + b[...][None, :]
        m = jnp.max(z, axis=1, keepdims=True)
        e = jnp.exp(z - m)
        out[...] = e / jnp.sum(e, axis=1, keepdims=True)

    return pl.pallas_call(
        body,
        grid=(N_PAD // _BLK,),
        in_specs=[
            pl.BlockSpec((NC, _BLK, D), lambda i: (0, i, 0)),
            pl.BlockSpec((_BLK, D), lambda i: (i, 0)),
            pl.BlockSpec((_BLK,), lambda i: (i,)),
            pl.BlockSpec((D,), lambda i: (0,)),
        ],
        out_specs=pl.BlockSpec((_BLK, D), lambda i: (i, 0)),
        out_shape=jax.ShapeDtypeStruct((N_PAD, D), jnp.float32),
    )(s_parts, g2, dinv, b2)


@jax.jit
def kernel(x, edge_index, W1, b1, W2, b2):
    n_edges = edge_index.shape[1]
    n_chunks = -(-n_edges // (NW * K))
    e_pad = NW * K * n_chunks

    src = edge_index[0].astype(jnp.int32)
    dst = edge_index[1].astype(jnp.int32)
    pad = e_pad - n_edges
    src_pad = jnp.concatenate([src, jnp.zeros((pad,), jnp.int32)])
    dst_pad = jnp.concatenate([dst, jnp.full((pad,), DUMMY_DST, jnp.int32)])
    x_pad = jnp.concatenate(
        [x, jnp.zeros((N_PAD - N_NODES, D), x.dtype)], axis=0)

    ones_blk = jnp.concatenate(
        [jnp.ones((K, 1), jnp.float32), jnp.zeros((K, DEG_W - 1), jnp.float32)],
        axis=1)
    zeros_deg = jnp.zeros((ROWS_PER_TILE, DEG_W), jnp.float32)
    zeros_row = jnp.zeros((ROWS_PER_TILE, D), jnp.float32)

    deg_parts = _sc_degree(dst_pad, ones_blk, zeros_deg, n_chunks)
    g1, dinv = _tc_prologue(deg_parts, x_pad, W1)
    s1 = _sc_scatter(src_pad, dst_pad, g1, zeros_row, n_chunks)
    g2 = _tc_mid(s1, g1, dinv, b1, W2)
    s2 = _sc_scatter(src_pad, dst_pad, g2, zeros_row, n_chunks)
    out = _tc_epilogue(s2, g2, dinv, b2)
    return out[:N_NODES]
